# baseline (speedup 1.0000x reference)
import functools

import jax
import jax.numpy as jnp
from jax import lax
from jax.experimental import pallas as pl
from jax.experimental.pallas import tpu as pltpu
from jax.experimental.pallas import tpu_sc as plsc

N = 10000
D = 128
E = 320000

NC = 2
NS = 16
CH = 128
CPT = 80
EPT = CH * CPT
EPAD = EPT * NC * NS
NACC = 10240
RPS = NACC // NS
CG = 64
CPG = EPT // CG
RING = 5
NCA = 1
EPT1 = EPAD // (NCA * NS)
CPG1 = EPT1 // CG

_MESH = plsc.VectorSubcoreMesh(
    core_axis_name="c", subcore_axis_name="s", num_cores=NC, num_subcores=NS
)


@functools.partial(
    pl.kernel,
    out_type=[jax.ShapeDtypeStruct((NC, NACC, D), jnp.float32)],
    mesh=_MESH,
    scratch_types=[
        pltpu.VMEM_SHARED((NACC, D), jnp.float32),
        pltpu.VMEM((CH,), jnp.int32),
        pltpu.VMEM((CH,), jnp.int32),
        pltpu.VMEM((CH, D), jnp.float32),
        pltpu.SemaphoreType.DMA,
        pltpu.SemaphoreType.DMA,
    ],
)
def _sc_counts(dst_hbm, z128_hbm, ones_hbm, c_out,
               cacc, dst_v0, dst_v1, ones_v, sd0, sd1):
    c = lax.axis_index("c")
    s = lax.axis_index("s")
    wid = c * NS + s
    dst_v = (dst_v0, dst_v1)
    sd = (sd0, sd1)
    for k in range(RPS // CH):
        pltpu.sync_copy(z128_hbm, cacc.at[pl.ds(s * RPS + k * CH, CH)])
    pltpu.sync_copy(ones_hbm, ones_v)
    plsc.subcore_barrier()

    def idx_load(j, b):
        pltpu.async_copy(dst_hbm.at[pl.ds(wid * EPT + j * CH, CH)], dst_v[b], sd[b])

    def idx_wait(b):
        pltpu.make_async_copy(dst_hbm.at[pl.ds(0, CH)], dst_v[b], sd[b]).wait()

    def step(j, b, prefetch):
        idx_wait(b)
        pltpu.sync_copy(ones_v, cacc.at[dst_v[b]], add=True)
        if prefetch:
            idx_load(j + 2, b)

    idx_load(0, 0)
    idx_load(1, 1)

    def pair(t, carry):
        step(2 * t, 0, True)
        step(2 * t + 1, 1, True)
        return carry

    lax.fori_loop(0, CPT // 2 - 1, pair, 0)
    step(CPT - 2, 0, False)
    step(CPT - 1, 1, False)
    plsc.subcore_barrier()
    pltpu.sync_copy(cacc.at[pl.ds(s * RPS, RPS)], c_out.at[c, pl.ds(s * RPS, RPS)])


@functools.partial(
    pl.kernel,
    out_type=[jax.ShapeDtypeStruct((NCA, NACC, D), jnp.float32)],
    mesh=plsc.VectorSubcoreMesh(
        core_axis_name="c", subcore_axis_name="s", num_cores=1, num_subcores=NS),
    scratch_types=(
        [pltpu.VMEM_SHARED((NACC, D), jnp.float32)]
        + [pltpu.VMEM((CG,), jnp.int32) for _ in range(2 * RING)]
        + [pltpu.VMEM((CG, D), jnp.float32) for _ in range(RING)]
        + [pltpu.SemaphoreType.DMA for _ in range(3 * RING)]
    ),
)
def _sc_agg(h_hbm, src_hbm, dst_hbm, z128_hbm, p_out, acc, *bufs):
    c = lax.axis_index("c")
    s = lax.axis_index("s")
    wid = c * NS + s
    src_v = bufs[0:RING]
    dst_v = bufs[RING:2 * RING]
    rows_v = bufs[2 * RING:3 * RING]
    ss = bufs[3 * RING:4 * RING]
    sd = bufs[4 * RING:5 * RING]
    sg = bufs[5 * RING:6 * RING]
    for k in range(RPS // CH):
        pltpu.sync_copy(z128_hbm, acc.at[pl.ds(s * RPS + k * CH, CH)])
    plsc.subcore_barrier()

    def idx_load(j, b):
        base = wid * EPT1 + j * CG
        pltpu.async_copy(src_hbm.at[pl.ds(base, CG)], src_v[b], ss[b])
        pltpu.async_copy(dst_hbm.at[pl.ds(base, CG)], dst_v[b], sd[b])

    def fire_gather(b):
        pltpu.make_async_copy(src_hbm.at[pl.ds(0, CG)], src_v[b], ss[b]).wait()
        pltpu.async_copy(h_hbm.at[src_v[b]], rows_v[b], sg[b])

    def step(j, b, gather_ahead, prefetch):
        if gather_ahead:
            fire_gather((b + RING - 1) % RING)
        pltpu.make_async_copy(h_hbm.at[pl.ds(0, CG)], rows_v[b], sg[b]).wait()
        pltpu.make_async_copy(dst_hbm.at[pl.ds(0, CG)], dst_v[b], sd[b]).wait()
        pltpu.sync_copy(rows_v[b], acc.at[dst_v[b]], add=True)
        if prefetch:
            idx_load(j + RING, b)

    for b in range(RING):
        idx_load(b, b)
    for b in range(RING - 1):
        fire_gather(b)

    def block(t, carry):
        for b in range(RING):
            step(t * RING + b, b, True, True)
        return carry

    lax.fori_loop(0, CPG1 // RING - 1, block, 0)
    for b in range(RING):
        j = CPG1 - RING + b
        step(j, b, j + RING - 1 < CPG1, False)
    plsc.subcore_barrier()
    pltpu.sync_copy(acc.at[pl.ds(s * RPS, RPS)], p_out.at[c, pl.ds(s * RPS, RPS)])


_BLK = 1000


def _tc_layer_body(relu, p_ref, c_ref, x_ref, wl_ref, bl_ref, wr_ref, o_ref):
    psum = jnp.sum(p_ref[...], axis=0)
    cnt = c_ref[0, :, 0:1] + c_ref[1, :, 0:1]
    agg = psum / jnp.maximum(cnt, 1.0)
    h = (jnp.dot(agg, wl_ref[...], preferred_element_type=jnp.float32)
         + bl_ref[...]
         + jnp.dot(x_ref[...], wr_ref[...], preferred_element_type=jnp.float32))
    o_ref[...] = jnp.maximum(h, 0.0) if relu else h


def _tc_layer(p, cnts, x, wl, bl, wr, relu):
    return pl.pallas_call(
        functools.partial(_tc_layer_body, relu),
        grid=(N // _BLK,),
        in_specs=[
            pl.BlockSpec((NCA, _BLK, D), lambda i: (0, i, 0)),
            pl.BlockSpec((NC, _BLK, D), lambda i: (0, i, 0)),
            pl.BlockSpec((_BLK, D), lambda i: (i, 0)),
            pl.BlockSpec((D, D), lambda i: (0, 0)),
            pl.BlockSpec((1, D), lambda i: (0, 0)),
            pl.BlockSpec((D, D), lambda i: (0, 0)),
        ],
        out_specs=pl.BlockSpec((_BLK, D), lambda i: (i, 0)),
        out_shape=jax.ShapeDtypeStruct((N, D), jnp.float32),
    )(p, cnts, x, wl, bl.reshape(1, D), wr)


def _tc_final_body(p_ref, c_ref, x_ref, wl_ref, bl_ref, wr_ref,
                   p1_ref, pb1_ref, p2_ref, pb2_ref, o_ref):
    psum = jnp.sum(p_ref[...], axis=0)
    cnt = c_ref[0, :, 0:1] + c_ref[1, :, 0:1]
    agg = psum / jnp.maximum(cnt, 1.0)
    h = (jnp.dot(agg, wl_ref[...], preferred_element_type=jnp.float32)
         + bl_ref[...]
         + jnp.dot(x_ref[...], wr_ref[...], preferred_element_type=jnp.float32))
    z = jnp.maximum(
        jnp.dot(h, p1_ref[...], preferred_element_type=jnp.float32) + pb1_ref[...],
        0.0)
    t = jnp.dot(z, p2_ref[...], preferred_element_type=jnp.float32) + pb2_ref[...]
    o_ref[...] = 1.0 / (1.0 + jnp.exp(-t))


def _tc_final(p, cnts, x, wl, bl, wr, P1, pb1, P2, pb2):
    return pl.pallas_call(
        _tc_final_body,
        grid=(N // _BLK,),
        in_specs=[
            pl.BlockSpec((NCA, _BLK, D), lambda i: (0, i, 0)),
            pl.BlockSpec((NC, _BLK, D), lambda i: (0, i, 0)),
            pl.BlockSpec((_BLK, D), lambda i: (i, 0)),
            pl.BlockSpec((D, D), lambda i: (0, 0)),
            pl.BlockSpec((1, D), lambda i: (0, 0)),
            pl.BlockSpec((D, D), lambda i: (0, 0)),
            pl.BlockSpec((D, D), lambda i: (0, 0)),
            pl.BlockSpec((1, D), lambda i: (0, 0)),
            pl.BlockSpec((D, 1), lambda i: (0, 0)),
            pl.BlockSpec((1, 1), lambda i: (0, 0)),
        ],
        out_specs=pl.BlockSpec((_BLK, 1), lambda i: (i, 0)),
        out_shape=jax.ShapeDtypeStruct((N, 1), jnp.float32),
    )(p, cnts, x, wl, bl.reshape(1, D), wr,
      P1, pb1.reshape(1, D), P2, pb2.reshape(1, 1))


def kernel(x, edge_index, W_l1, b_l1, W_r1, W_l2, b_l2, W_r2,
           W_l3, b_l3, W_r3, P1, pb1, P2, pb2):
    src = edge_index[0]
    dst = edge_index[1]
    npad = EPAD - E
    src1 = jnp.concatenate([src, jnp.zeros((npad,), jnp.int32)])
    dst1 = jnp.concatenate([dst, jnp.full((npad,), N, jnp.int32)])
    z128 = jnp.zeros((CH, D), jnp.float32)
    ones128 = jnp.ones((CH, D), jnp.float32)

    (cnts,) = _sc_counts(dst1, z128, ones128)
    (p1,) = _sc_agg(x, src1, dst1, z128)
    h1 = _tc_layer(p1, cnts, x, W_l1, b_l1, W_r1, relu=True)
    (p2,) = _sc_agg(h1, src1, dst1, z128)
    h2 = _tc_layer(p2, cnts, h1, W_l2, b_l2, W_r2, relu=True)
    (p3,) = _sc_agg(h2, src1, dst1, z128)
    return _tc_final(p3, cnts, h2, W_l3, b_l3, W_r3, P1, pb1, P2, pb2)


# speedup vs baseline: 1.1962x; 1.1962x over previous
import functools

import jax
import jax.numpy as jnp
from jax import lax
from jax.experimental import pallas as pl
from jax.experimental.pallas import tpu as pltpu
from jax.experimental.pallas import tpu_sc as plsc

N = 10000
D = 128
E = 320000

NC = 2
NS = 16
CH = 128
CPT = 80
EPT = CH * CPT
EPAD = EPT * NC * NS
NACC = 10240
RPS = NACC // NS

_MESH = plsc.VectorSubcoreMesh(
    core_axis_name="c", subcore_axis_name="s", num_cores=NC, num_subcores=NS
)


@functools.partial(
    pl.kernel,
    out_type=[jax.ShapeDtypeStruct((NC, NACC, D), jnp.float32)],
    mesh=_MESH,
    scratch_types=[
        pltpu.VMEM_SHARED((NACC, D), jnp.float32),
        pltpu.VMEM((CH,), jnp.int32),
        pltpu.VMEM((CH,), jnp.int32),
        pltpu.VMEM((CH, D), jnp.float32),
        pltpu.SemaphoreType.DMA,
        pltpu.SemaphoreType.DMA,
    ],
)
def _sc_counts(dst_hbm, z128_hbm, ones_hbm, c_out,
               cacc, dst_v0, dst_v1, ones_v, sd0, sd1):
    c = lax.axis_index("c")
    s = lax.axis_index("s")
    wid = c * NS + s
    dst_v = (dst_v0, dst_v1)
    sd = (sd0, sd1)
    for k in range(RPS // CH):
        pltpu.sync_copy(z128_hbm, cacc.at[pl.ds(s * RPS + k * CH, CH)])
    pltpu.sync_copy(ones_hbm, ones_v)
    plsc.subcore_barrier()

    def idx_load(j, b):
        pltpu.async_copy(dst_hbm.at[pl.ds(wid * EPT + j * CH, CH)], dst_v[b], sd[b])

    def idx_wait(b):
        pltpu.make_async_copy(dst_hbm.at[pl.ds(0, CH)], dst_v[b], sd[b]).wait()

    def step(j, b, prefetch):
        idx_wait(b)
        pltpu.sync_copy(ones_v, cacc.at[dst_v[b]], add=True)
        if prefetch:
            idx_load(j + 2, b)

    idx_load(0, 0)
    idx_load(1, 1)

    def pair(t, carry):
        step(2 * t, 0, True)
        step(2 * t + 1, 1, True)
        return carry

    lax.fori_loop(0, CPT // 2 - 1, pair, 0)
    step(CPT - 2, 0, False)
    step(CPT - 1, 1, False)
    plsc.subcore_barrier()
    pltpu.sync_copy(cacc.at[pl.ds(s * RPS, RPS)], c_out.at[c, pl.ds(s * RPS, RPS)])


@functools.partial(
    pl.kernel,
    out_type=[jax.ShapeDtypeStruct((NC, NACC, D), jnp.float32)],
    mesh=_MESH,
    scratch_types=[
        pltpu.VMEM_SHARED((NACC, D), jnp.float32),
        pltpu.VMEM((CH,), jnp.int32),
        pltpu.VMEM((CH,), jnp.int32),
        pltpu.VMEM((CH,), jnp.int32),
        pltpu.VMEM((CH,), jnp.int32),
        pltpu.VMEM((CH, D), jnp.float32),
        pltpu.VMEM((CH, D), jnp.float32),
        pltpu.SemaphoreType.DMA,
        pltpu.SemaphoreType.DMA,
        pltpu.SemaphoreType.DMA,
        pltpu.SemaphoreType.DMA,
        pltpu.SemaphoreType.DMA,
        pltpu.SemaphoreType.DMA,
    ],
)
def _sc_agg(h_hbm, src_hbm, dst_hbm, z128_hbm, p_out, acc,
            src_v0, src_v1, dst_v0, dst_v1, rows_v0, rows_v1,
            ss0, ss1, sd0, sd1, sg0, sg1):
    c = lax.axis_index("c")
    s = lax.axis_index("s")
    wid = c * NS + s
    src_v = (src_v0, src_v1)
    dst_v = (dst_v0, dst_v1)
    rows_v = (rows_v0, rows_v1)
    ss = (ss0, ss1)
    sd = (sd0, sd1)
    sg = (sg0, sg1)
    for k in range(RPS // CH):
        pltpu.sync_copy(z128_hbm, acc.at[pl.ds(s * RPS + k * CH, CH)])
    plsc.subcore_barrier()

    def idx_load(j, b):
        base = wid * EPT + j * CH
        pltpu.async_copy(src_hbm.at[pl.ds(base, CH)], src_v[b], ss[b])
        pltpu.async_copy(dst_hbm.at[pl.ds(base, CH)], dst_v[b], sd[b])

    def step(j, b, gather_next, prefetch):
        nb = 1 - b
        if gather_next:
            pltpu.make_async_copy(src_hbm.at[pl.ds(0, CH)], src_v[nb], ss[nb]).wait()
            pltpu.async_copy(h_hbm.at[src_v[nb]], rows_v[nb], sg[nb])
        pltpu.make_async_copy(h_hbm.at[pl.ds(0, CH)], rows_v[b], sg[b]).wait()
        pltpu.make_async_copy(dst_hbm.at[pl.ds(0, CH)], dst_v[b], sd[b]).wait()
        pltpu.sync_copy(rows_v[b], acc.at[dst_v[b]], add=True)
        if prefetch:
            idx_load(j + 2, b)

    idx_load(0, 0)
    idx_load(1, 1)
    pltpu.make_async_copy(src_hbm.at[pl.ds(0, CH)], src_v[0], ss[0]).wait()
    pltpu.async_copy(h_hbm.at[src_v[0]], rows_v[0], sg[0])

    def pair(t, carry):
        step(2 * t, 0, True, True)
        step(2 * t + 1, 1, True, True)
        return carry

    lax.fori_loop(0, CPT // 2 - 1, pair, 0)
    step(CPT - 2, 0, True, False)
    step(CPT - 1, 1, False, False)
    plsc.subcore_barrier()
    pltpu.sync_copy(acc.at[pl.ds(s * RPS, RPS)], p_out.at[c, pl.ds(s * RPS, RPS)])


_BLK = 1000


def _tc_layer_body(relu, p_ref, c_ref, x_ref, wl_ref, bl_ref, wr_ref, o_ref):
    psum = p_ref[0] + p_ref[1]
    cnt = c_ref[0, :, 0:1] + c_ref[1, :, 0:1]
    agg = psum / jnp.maximum(cnt, 1.0)
    h = (jnp.dot(agg, wl_ref[...], preferred_element_type=jnp.float32)
         + bl_ref[...]
         + jnp.dot(x_ref[...], wr_ref[...], preferred_element_type=jnp.float32))
    o_ref[...] = jnp.maximum(h, 0.0) if relu else h


def _tc_layer(p, cnts, x, wl, bl, wr, relu):
    return pl.pallas_call(
        functools.partial(_tc_layer_body, relu),
        grid=(N // _BLK,),
        in_specs=[
            pl.BlockSpec((NC, _BLK, D), lambda i: (0, i, 0)),
            pl.BlockSpec((NC, _BLK, D), lambda i: (0, i, 0)),
            pl.BlockSpec((_BLK, D), lambda i: (i, 0)),
            pl.BlockSpec((D, D), lambda i: (0, 0)),
            pl.BlockSpec((1, D), lambda i: (0, 0)),
            pl.BlockSpec((D, D), lambda i: (0, 0)),
        ],
        out_specs=pl.BlockSpec((_BLK, D), lambda i: (i, 0)),
        out_shape=jax.ShapeDtypeStruct((N, D), jnp.float32),
    )(p, cnts, x, wl, bl.reshape(1, D), wr)


def _tc_final_body(p_ref, c_ref, x_ref, wl_ref, bl_ref, wr_ref,
                   p1_ref, pb1_ref, p2_ref, pb2_ref, o_ref):
    psum = p_ref[0] + p_ref[1]
    cnt = c_ref[0, :, 0:1] + c_ref[1, :, 0:1]
    agg = psum / jnp.maximum(cnt, 1.0)
    h = (jnp.dot(agg, wl_ref[...], preferred_element_type=jnp.float32)
         + bl_ref[...]
         + jnp.dot(x_ref[...], wr_ref[...], preferred_element_type=jnp.float32))
    z = jnp.maximum(
        jnp.dot(h, p1_ref[...], preferred_element_type=jnp.float32) + pb1_ref[...],
        0.0)
    t = jnp.dot(z, p2_ref[...], preferred_element_type=jnp.float32) + pb2_ref[...]
    o_ref[...] = 1.0 / (1.0 + jnp.exp(-t))


def _tc_final(p, cnts, x, wl, bl, wr, P1, pb1, P2, pb2):
    return pl.pallas_call(
        _tc_final_body,
        grid=(N // _BLK,),
        in_specs=[
            pl.BlockSpec((NC, _BLK, D), lambda i: (0, i, 0)),
            pl.BlockSpec((NC, _BLK, D), lambda i: (0, i, 0)),
            pl.BlockSpec((_BLK, D), lambda i: (i, 0)),
            pl.BlockSpec((D, D), lambda i: (0, 0)),
            pl.BlockSpec((1, D), lambda i: (0, 0)),
            pl.BlockSpec((D, D), lambda i: (0, 0)),
            pl.BlockSpec((D, D), lambda i: (0, 0)),
            pl.BlockSpec((1, D), lambda i: (0, 0)),
            pl.BlockSpec((D, 1), lambda i: (0, 0)),
            pl.BlockSpec((1, 1), lambda i: (0, 0)),
        ],
        out_specs=pl.BlockSpec((_BLK, 1), lambda i: (i, 0)),
        out_shape=jax.ShapeDtypeStruct((N, 1), jnp.float32),
    )(p, cnts, x, wl, bl.reshape(1, D), wr,
      P1, pb1.reshape(1, D), P2, pb2.reshape(1, 1))


def kernel(x, edge_index, W_l1, b_l1, W_r1, W_l2, b_l2, W_r2,
           W_l3, b_l3, W_r3, P1, pb1, P2, pb2):
    src = edge_index[0]
    dst = edge_index[1]
    npad = EPAD - E
    src1 = jnp.concatenate([src, jnp.zeros((npad,), jnp.int32)])
    dst1 = jnp.concatenate([dst, jnp.full((npad,), N, jnp.int32)])
    z128 = jnp.zeros((CH, D), jnp.float32)
    ones128 = jnp.ones((CH, D), jnp.float32)

    (cnts,) = _sc_counts(dst1, z128, ones128)
    (p1,) = _sc_agg(x, src1, dst1, z128)
    h1 = _tc_layer(p1, cnts, x, W_l1, b_l1, W_r1, relu=True)
    (p2,) = _sc_agg(h1, src1, dst1, z128)
    h2 = _tc_layer(p2, cnts, h1, W_l2, b_l2, W_r2, relu=True)
    (p3,) = _sc_agg(h2, src1, dst1, z128)
    return _tc_final(p3, cnts, h2, W_l3, b_l3, W_r3, P1, pb1, P2, pb2)
